# baseline (speedup 1.0000x reference)
import functools

import jax
import jax.numpy as jnp
from jax.experimental import pallas as pl
from jax.experimental.pallas import tpu as pltpu

_K = 10
_N = 8192
_D = 256
_BR = 256
_NBLK = _N // _BR
_NLANE = 128
_NCOL = _N // _NLANE
_NPAN = 8
_GPP = _NCOL // _NPAN
_PW = _N // _NPAN


def _topk_slow(s):
    col_ids = jax.lax.broadcasted_iota(jnp.int32, s.shape, 1)
    picked = []
    for _ in range(_K):
        idx = jnp.argmax(s, axis=1).astype(jnp.int32)[:, None]
        picked.append(idx)
        s = jnp.where(col_ids == idx, -jnp.inf, s)
    return jnp.concatenate(picked, axis=1)


def _block_topk(xn_ref, idx_ref, base, sub):
    xn_blk = xn_ref[pl.ds(base, _BR), :]

    lane = jax.lax.broadcasted_iota(jnp.int32, (_BR, _NLANE), 1)
    neg = jnp.full((_BR, _NLANE), -jnp.inf, jnp.float32)
    zero = jnp.zeros((_BR, _NLANE), jnp.int32)
    c1, c2, c3, c4 = neg, neg, neg, neg
    g1, g2, g3 = zero, zero, zero

    panels = []
    for p in range(_NPAN):
        xn_pan = xn_ref[p * _PW:(p + 1) * _PW, :]
        sp = jax.lax.dot_general(
            xn_blk, xn_pan, (((1,), (1,)), ((), ())),
            preferred_element_type=jnp.float32)
        panels.append(sp)
        for gl in range(_GPP):
            g = p * _GPP + gl
            v = sp[:, gl * _NLANE:(gl + 1) * _NLANE]
            gq = jnp.full((_BR, _NLANE), g, jnp.int32)
            b1 = v > c1
            b2 = v > c2
            b3 = v > c3
            b4 = v > c4
            c4 = jnp.where(b3, c3, jnp.where(b4, v, c4))
            c3 = jnp.where(b2, c2, jnp.where(b3, v, c3))
            g3 = jnp.where(b2, g2, jnp.where(b3, gq, g3))
            c2 = jnp.where(b1, c1, jnp.where(b2, v, c2))
            g2 = jnp.where(b1, g1, jnp.where(b2, gq, g2))
            c1 = jnp.where(b1, v, c1)
            g1 = jnp.where(b1, gq, g1)

    i1 = g1 * _NLANE + lane
    i2 = g2 * _NLANE + lane
    i3 = g3 * _NLANE + lane

    picked = []
    m = None
    for _ in range(_K):
        m = jnp.max(c1, axis=1, keepdims=True)
        idx = jnp.min(jnp.where(c1 == m, i1, _N), axis=1, keepdims=True)
        picked.append(idx)
        hit = i1 == idx
        c1 = jnp.where(hit, c2, c1)
        i1 = jnp.where(hit, i2, i1)
        c2 = jnp.where(hit, c3, c2)
        i2 = jnp.where(hit, i3, i2)
        c3 = jnp.where(hit, -jnp.inf, c3)
    idx_ref[sub * _BR:(sub + 1) * _BR, :] = jnp.concatenate(picked, axis=1)

    c4max = jnp.max(c4, axis=1, keepdims=True)
    nbad = jnp.sum(jnp.where(c4max >= m, 1.0, 0.0))

    @pl.when(nbad > 0.0)
    def _fallback():
        idx_ref[sub * _BR:(sub + 1) * _BR, :] = _topk_slow(
            jnp.concatenate(panels, axis=1))


def _knn_kernel(x_ref, idx_ref, val_ref, xn_ref):
    t = pl.program_id(0)

    @pl.when(t == 0)
    def _normalize():
        x = x_ref[...]
        n2 = jnp.sum(x * x, axis=1, keepdims=True)
        xn_ref[...] = x / jnp.sqrt(n2)

    for sub in range(2):
        base = pl.multiple_of((2 * t + sub) * _BR, _BR)
        _block_topk(xn_ref, idx_ref, base, sub)

    row_sum = jnp.float32(1e-7) + jnp.float32(_K)
    r_inv_sqrt = row_sum ** -0.5
    val_ref[...] = jnp.full((2 * _BR, _K), r_inv_sqrt * r_inv_sqrt, jnp.float32)


@functools.partial(jax.jit)
def kernel(mm_embedding):
    knn_ind, vals = pl.pallas_call(
        _knn_kernel,
        grid=(_NBLK // 2,),
        in_specs=[pl.BlockSpec((_N, _D), lambda t: (0, 0))],
        out_specs=[
            pl.BlockSpec((2 * _BR, _K), lambda t: (t, 0)),
            pl.BlockSpec((2 * _BR, _K), lambda t: (t, 0)),
        ],
        out_shape=[
            jax.ShapeDtypeStruct((_N, _K), jnp.int32),
            jax.ShapeDtypeStruct((_N, _K), jnp.float32),
        ],
        scratch_shapes=[pltpu.VMEM((_N, _D), jnp.float32)],
    )(mm_embedding)

    rows = jnp.broadcast_to(jnp.arange(_N)[:, None], (_N, _K)).reshape(-1)
    indices = jnp.stack((rows, knn_ind.reshape(-1)), axis=0)
    return (indices, vals.reshape(-1))


# speedup vs baseline: 1.6826x; 1.6826x over previous
import functools

import jax
import jax.numpy as jnp
from jax.experimental import pallas as pl
from jax.experimental.pallas import tpu as pltpu

_K = 10
_N = 8192
_D = 256
_BR = 256
_NBLK = _N // _BR
_NLANE = 128
_NCOL = _N // _NLANE
_NPAN = 16
_GPP = _NCOL // _NPAN
_PW = _N // _NPAN


def _topk_slow(s):
    col_ids = jax.lax.broadcasted_iota(jnp.int32, s.shape, 1)
    picked = []
    for _ in range(_K):
        idx = jnp.argmax(s, axis=1).astype(jnp.int32)[:, None]
        picked.append(idx)
        s = jnp.where(col_ids == idx, -jnp.inf, s)
    return jnp.concatenate(picked, axis=1)


def _knn_kernel(x_ref, idx_ref, val_ref, xn_ref):
    i = pl.program_id(0)

    @pl.when(i == 0)
    def _normalize():
        x = x_ref[...]
        n2 = jnp.sum(x * x, axis=1, keepdims=True)
        xn_ref[...] = x / jnp.sqrt(n2)

    base = pl.multiple_of(i * _BR, _BR)
    xn_blk = xn_ref[pl.ds(base, _BR), :]

    lane = jax.lax.broadcasted_iota(jnp.int32, (_BR, _NLANE), 1)
    neg = jnp.full((_BR, _NLANE), -jnp.inf, jnp.float32)
    zero = jnp.zeros((_BR, _NLANE), jnp.int32)
    c1, c2, c3, c4 = neg, neg, neg, neg
    g1, g2, g3 = zero, zero, zero

    panels = []
    for p in range(_NPAN):
        xn_pan = xn_ref[p * _PW:(p + 1) * _PW, :]
        sp = jax.lax.dot_general(
            xn_blk, xn_pan, (((1,), (1,)), ((), ())),
            preferred_element_type=jnp.float32)
        panels.append(sp)
        for gl in range(_GPP):
            g = p * _GPP + gl
            v = sp[:, gl * _NLANE:(gl + 1) * _NLANE]
            gq = jnp.full((_BR, _NLANE), g, jnp.int32)
            b1 = v > c1
            b2 = v > c2
            b3 = v > c3
            b4 = v > c4
            c4 = jnp.where(b3, c3, jnp.where(b4, v, c4))
            c3 = jnp.where(b2, c2, jnp.where(b3, v, c3))
            g3 = jnp.where(b2, g2, jnp.where(b3, gq, g3))
            c2 = jnp.where(b1, c1, jnp.where(b2, v, c2))
            g2 = jnp.where(b1, g1, jnp.where(b2, gq, g2))
            c1 = jnp.where(b1, v, c1)
            g1 = jnp.where(b1, gq, g1)

    i1 = g1 * _NLANE + lane
    i2 = g2 * _NLANE + lane
    i3 = g3 * _NLANE + lane

    picked = []
    m = None
    for _ in range(_K):
        m = jnp.max(c1, axis=1, keepdims=True)
        idx = jnp.min(jnp.where(c1 == m, i1, _N), axis=1, keepdims=True)
        picked.append(idx)
        hit = i1 == idx
        c1 = jnp.where(hit, c2, c1)
        i1 = jnp.where(hit, i2, i1)
        c2 = jnp.where(hit, c3, c2)
        i2 = jnp.where(hit, i3, i2)
        c3 = jnp.where(hit, -jnp.inf, c3)
    idx_ref[...] = jnp.concatenate(picked, axis=1)

    c4max = jnp.max(c4, axis=1, keepdims=True)
    nbad = jnp.sum(jnp.where(c4max >= m, 1.0, 0.0))

    @pl.when(nbad > 0.0)
    def _fallback():
        idx_ref[...] = _topk_slow(jnp.concatenate(panels, axis=1))

    row_sum = jnp.float32(1e-7) + jnp.float32(_K)
    r_inv_sqrt = row_sum ** -0.5
    val_ref[...] = jnp.full((_BR, _K), r_inv_sqrt * r_inv_sqrt, jnp.float32)


@functools.partial(jax.jit)
def kernel(mm_embedding):
    knn_ind, vals = pl.pallas_call(
        _knn_kernel,
        grid=(_NBLK,),
        in_specs=[pl.BlockSpec((_N, _D), lambda i: (0, 0))],
        out_specs=[
            pl.BlockSpec((_BR, _K), lambda i: (i, 0)),
            pl.BlockSpec((_BR, _K), lambda i: (i, 0)),
        ],
        out_shape=[
            jax.ShapeDtypeStruct((_N, _K), jnp.int32),
            jax.ShapeDtypeStruct((_N, _K), jnp.float32),
        ],
        scratch_shapes=[pltpu.VMEM((_N, _D), jnp.float32)],
    )(mm_embedding)

    rows = jnp.broadcast_to(jnp.arange(_N)[:, None], (_N, _K)).reshape(-1)
    indices = jnp.stack((rows, knn_ind.reshape(-1)), axis=0)
    return (indices, vals.reshape(-1))
